# Initial kernel scaffold; baseline (speedup 1.0000x reference)
import jax
import jax.numpy as jnp
from jax.experimental import pallas as pl


def kernel(x, edge_index, W1, b1, W2, b2):
    raise NotImplementedError("write your pallas kernel here")


# speedup vs baseline: 27.2464x; 27.2464x over previous
import functools

import jax
import jax.numpy as jnp
from jax import lax
from jax.experimental import pallas as pl
from jax.experimental.pallas import tpu as pltpu
from jax.experimental.pallas import tpu_sc as plsc

N_NODES = 10000
N_PAD = 10240
ROWS = N_PAD // 128
E_EDGES = 320000
NUM_CORES = 2
NUM_SUBCORES = 16
NUM_WORKERS = NUM_CORES * NUM_SUBCORES
CHUNK = 128
CPW = 79
E_PAD = NUM_WORKERS * CPW * CHUNK
LANES = 16
F32 = jnp.float32

_MESH = plsc.VectorSubcoreMesh(core_axis_name="c", subcore_axis_name="s")


def _zero_vmem(buf, n):
    def body(i, _):
        buf[pl.ds(i * LANES, LANES)] = jnp.zeros((LANES,), F32)
        return 0
    lax.fori_loop(0, n // LANES, body, 0)


@functools.partial(
    pl.kernel,
    out_type=(
        jax.ShapeDtypeStruct((NUM_CORES, N_PAD), F32),
        jax.ShapeDtypeStruct((NUM_CORES, N_PAD), F32),
    ),
    mesh=_MESH,
    scratch_types=(
        pltpu.VMEM((CPW, CHUNK), jnp.int32),
        pltpu.VMEM((CHUNK,), F32),
        pltpu.VMEM((N_PAD,), F32),
        pltpu.VMEM_SHARED((N_PAD,), F32),
        pltpu.VMEM_SHARED((N_PAD,), F32),
    ),
)
def _deg_kernel(src_h, dst_h, dop_h, dip_h, idx_v, ones_v, buf_v, acc_o, acc_i):
    c = lax.axis_index("c")
    s = lax.axis_index("s")
    wid = c * NUM_SUBCORES + s

    def fill_ones(i, _):
        ones_v[pl.ds(i * LANES, LANES)] = jnp.ones((LANES,), F32)
        return 0
    lax.fori_loop(0, CHUNK // LANES, fill_ones, 0)

    @pl.when(s == 0)
    def _():
        _zero_vmem(buf_v, N_PAD)
        pltpu.sync_copy(buf_v, acc_o)
        pltpu.sync_copy(buf_v, acc_i)
    plsc.subcore_barrier()

    pltpu.sync_copy(src_h.at[wid], idx_v)

    def scat_src(j, _):
        pltpu.sync_copy(ones_v, acc_o.at[idx_v.at[j]], add=True)
        return 0
    lax.fori_loop(0, CPW, scat_src, 0)

    pltpu.sync_copy(dst_h.at[wid], idx_v)

    def scat_dst(j, _):
        pltpu.sync_copy(ones_v, acc_i.at[idx_v.at[j]], add=True)
        return 0
    lax.fori_loop(0, CPW, scat_dst, 0)
    plsc.subcore_barrier()

    @pl.when(s == 0)
    def _():
        pltpu.sync_copy(acc_o, buf_v)
        pltpu.sync_copy(buf_v, dop_h.at[c])
        pltpu.sync_copy(acc_i, buf_v)
        pltpu.sync_copy(buf_v, dip_h.at[c])


@functools.partial(
    pl.kernel,
    out_type=jax.ShapeDtypeStruct((NUM_CORES, N_PAD), F32),
    mesh=_MESH,
    scratch_types=(
        pltpu.VMEM((CPW, CHUNK), jnp.int32),
        pltpu.VMEM((CPW, CHUNK), jnp.int32),
        pltpu.VMEM((CHUNK,), F32),
        pltpu.VMEM((N_PAD,), F32),
        pltpu.VMEM_SHARED((N_PAD,), F32),
        pltpu.VMEM_SHARED((N_PAD,), F32),
    ),
)
def _edge_kernel(src_h, dst_h, tbl_h, out_h, sidx_v, didx_v, vals_v, buf_v,
                 tbl_s, acc_s):
    c = lax.axis_index("c")
    s = lax.axis_index("s")
    wid = c * NUM_SUBCORES + s

    @pl.when(s == 0)
    def _():
        _zero_vmem(buf_v, N_PAD)
        pltpu.sync_copy(buf_v, acc_s)
        pltpu.sync_copy(tbl_h, buf_v)
        pltpu.sync_copy(buf_v, tbl_s)
    plsc.subcore_barrier()

    pltpu.sync_copy(src_h.at[wid], sidx_v)
    pltpu.sync_copy(dst_h.at[wid], didx_v)

    def step(j, _):
        pltpu.sync_copy(tbl_s.at[didx_v.at[j]], vals_v)
        pltpu.sync_copy(vals_v, acc_s.at[sidx_v.at[j]], add=True)
        return 0
    lax.fori_loop(0, CPW, step, 0)
    plsc.subcore_barrier()

    @pl.when(s == 0)
    def _():
        pltpu.sync_copy(acc_s, buf_v)
        pltpu.sync_copy(buf_v, out_h.at[c])


def _norms_body(dop_ref, dip_ref, ns_ref, nd_ref):
    do = dop_ref[0] + dop_ref[1]
    di = dip_ref[0] + dip_ref[1]
    ns_ref[...] = lax.rsqrt(jnp.maximum(do, 1.0))
    nd_ref[...] = lax.rsqrt(jnp.maximum(di, 1.0))


_norms_call = pl.pallas_call(
    _norms_body,
    out_shape=(
        jax.ShapeDtypeStruct((ROWS, 128), F32),
        jax.ShapeDtypeStruct((ROWS, 128), F32),
    ),
)


def _u_body(wp_ref, ns_ref, nd_ref, u_ref, sw_ref):
    w = ns_ref[...] * (wp_ref[0] + wp_ref[1])
    u_ref[...] = nd_ref[...] * w
    r = lax.broadcasted_iota(jnp.int32, (ROWS, 128), 0)
    col = lax.broadcasted_iota(jnp.int32, (ROWS, 128), 1)
    valid = (r * 128 + col) < N_NODES
    sw_ref[...] = jnp.sum(jnp.where(valid, w, 0.0)).reshape(1, 1)


_u_call = pl.pallas_call(
    _u_body,
    out_shape=(
        jax.ShapeDtypeStruct((ROWS, 128), F32),
        jax.ShapeDtypeStruct((1, 1), F32),
    ),
)


BLK = 2000
GRID = N_NODES // BLK


def _final_body(vp0_ref, vp1_ref, ns_ref, x_ref, w1_ref, b1_ref, w2_ref,
                b2_ref, sw_ref, out_ref, acc_ref):
    i = pl.program_id(0)

    @pl.when(i == 0)
    def _():
        acc_ref[...] = jnp.zeros_like(acc_ref)

    v = ns_ref[...] * (vp0_ref[...] + vp1_ref[...])
    acc_ref[...] += jnp.sum(v * x_ref[...], axis=0, keepdims=True)

    @pl.when(i == GRID - 1)
    def _():
        y = acc_ref[...]
        t = jnp.dot(y, w1_ref[...], preferred_element_type=F32)
        t = jnp.dot(t, w2_ref[...], preferred_element_type=F32)
        bias = sw_ref[...] * jnp.dot(b1_ref[...], w2_ref[...],
                                     preferred_element_type=F32)
        out_ref[...] = (t + bias) * (1.0 / N_NODES) + b2_ref[...]


_final_call = pl.pallas_call(
    _final_body,
    grid=(GRID,),
    in_specs=[
        pl.BlockSpec((BLK, 1), lambda i: (i, 0)),
        pl.BlockSpec((BLK, 1), lambda i: (i, 0)),
        pl.BlockSpec((BLK, 1), lambda i: (i, 0)),
        pl.BlockSpec((BLK, 128), lambda i: (i, 0)),
        pl.BlockSpec((128, 128), lambda i: (0, 0)),
        pl.BlockSpec((1, 128), lambda i: (0, 0)),
        pl.BlockSpec((128, 64), lambda i: (0, 0)),
        pl.BlockSpec((1, 64), lambda i: (0, 0)),
        pl.BlockSpec((1, 1), lambda i: (0, 0)),
    ],
    out_specs=pl.BlockSpec((1, 64), lambda i: (0, 0)),
    out_shape=jax.ShapeDtypeStruct((1, 64), F32),
    scratch_shapes=[pltpu.VMEM((1, 128), F32)],
)


def kernel(x, edge_index, W1, b1, W2, b2):
    src = edge_index[0]
    dst = edge_index[1]
    pad = jnp.full((E_PAD - E_EDGES,), N_NODES, jnp.int32)
    src_p = jnp.concatenate([src, pad]).reshape(NUM_WORKERS, CPW, CHUNK)
    dst_p = jnp.concatenate([dst, pad]).reshape(NUM_WORKERS, CPW, CHUNK)

    dop, dip = _deg_kernel(src_p, dst_p)
    ns, nd = _norms_call(dop.reshape(NUM_CORES, ROWS, 128),
                         dip.reshape(NUM_CORES, ROWS, 128))
    wp = _edge_kernel(src_p, dst_p, nd.reshape(N_PAD))
    u, sw = _u_call(wp.reshape(NUM_CORES, ROWS, 128), ns, nd)
    vp = _edge_kernel(src_p, dst_p, u.reshape(N_PAD))

    out = _final_call(
        vp[0].reshape(N_PAD, 1), vp[1].reshape(N_PAD, 1),
        ns.reshape(N_PAD, 1), x, W1, b1.reshape(1, -1), W2,
        b2.reshape(1, -1), sw)
    return out


# speedup vs baseline: 33.3144x; 1.2227x over previous
import functools

import jax
import jax.numpy as jnp
from jax import lax
from jax.experimental import pallas as pl
from jax.experimental.pallas import tpu as pltpu
from jax.experimental.pallas import tpu_sc as plsc

N_NODES = 10000
N_PAD = 10240
ROWS = N_PAD // 128
E_EDGES = 320000
NUM_CORES = 2
NUM_SUBCORES = 16
NUM_WORKERS = NUM_CORES * NUM_SUBCORES
CHUNK = 128
CPW = 79
E_PAD = NUM_WORKERS * CPW * CHUNK
LANES = 16
F32 = jnp.float32

_MESH = plsc.VectorSubcoreMesh(core_axis_name="c", subcore_axis_name="s")


@functools.partial(
    pl.kernel,
    out_type=(
        jax.ShapeDtypeStruct((NUM_CORES, N_PAD), F32),
        jax.ShapeDtypeStruct((NUM_CORES, N_PAD), F32),
    ),
    mesh=_MESH,
    scratch_types=(
        pltpu.VMEM((CPW, CHUNK), jnp.int32),
        pltpu.VMEM((CPW, CHUNK), jnp.int32),
        pltpu.VMEM((CHUNK,), F32),
        pltpu.VMEM((N_PAD,), F32),
        pltpu.VMEM_SHARED((N_PAD,), F32),
        pltpu.VMEM_SHARED((N_PAD,), F32),
        pltpu.SemaphoreType.DMA,
    ),
)
def _deg_kernel(src_h, dst_h, zeros_h, dop_h, dip_h, sidx_v, didx_v, ones_v,
                buf_v, acc_o, acc_i, ssem):
    c = lax.axis_index("c")
    s = lax.axis_index("s")
    wid = c * NUM_SUBCORES + s

    def fill_ones(i, _):
        ones_v[pl.ds(i * LANES, LANES)] = jnp.ones((LANES,), F32)
        return 0
    lax.fori_loop(0, CHUNK // LANES, fill_ones, 0)

    @pl.when(s == 0)
    def _():
        pltpu.sync_copy(zeros_h, buf_v)
        pltpu.sync_copy(buf_v, acc_o)
        pltpu.sync_copy(buf_v, acc_i)
    plsc.subcore_barrier()

    pltpu.sync_copy(src_h.at[wid], sidx_v)
    pltpu.sync_copy(dst_h.at[wid], didx_v)

    def scat_src(j, _):
        pltpu.async_copy(ones_v, acc_o.at[sidx_v.at[j]], ssem, add=True)
        return 0
    lax.fori_loop(0, CPW, scat_src, 0)

    def scat_dst(j, _):
        pltpu.async_copy(ones_v, acc_i.at[didx_v.at[j]], ssem, add=True)
        return 0
    lax.fori_loop(0, CPW, scat_dst, 0)

    def drain(j, _):
        pltpu.make_async_copy(ones_v, acc_o.at[sidx_v.at[0]], ssem).wait()
        return 0
    lax.fori_loop(0, 2 * CPW, drain, 0)
    plsc.subcore_barrier()

    @pl.when(s == 0)
    def _():
        pltpu.sync_copy(acc_o, buf_v)
        pltpu.sync_copy(buf_v, dop_h.at[c])
        pltpu.sync_copy(acc_i, buf_v)
        pltpu.sync_copy(buf_v, dip_h.at[c])


@functools.partial(
    pl.kernel,
    out_type=jax.ShapeDtypeStruct((NUM_CORES, N_PAD), F32),
    mesh=_MESH,
    scratch_types=(
        pltpu.VMEM((CPW, CHUNK), jnp.int32),
        pltpu.VMEM((CPW, CHUNK), jnp.int32),
        pltpu.VMEM((CPW, CHUNK), F32),
        pltpu.VMEM((N_PAD,), F32),
        pltpu.VMEM_SHARED((N_PAD,), F32),
        pltpu.VMEM_SHARED((N_PAD,), F32),
        pltpu.SemaphoreType.DMA,
        pltpu.SemaphoreType.DMA,
    ),
)
def _edge_kernel(src_h, dst_h, tbl_h, zeros_h, out_h, sidx_v, didx_v, vals_v,
                 buf_v, tbl_s, acc_s, gsem, ssem):
    c = lax.axis_index("c")
    s = lax.axis_index("s")
    wid = c * NUM_SUBCORES + s

    @pl.when(s == 0)
    def _():
        pltpu.sync_copy(zeros_h, buf_v)
        pltpu.sync_copy(buf_v, acc_s)
        pltpu.sync_copy(tbl_h, buf_v)
        pltpu.sync_copy(buf_v, tbl_s)
    plsc.subcore_barrier()

    pltpu.sync_copy(src_h.at[wid], sidx_v)
    pltpu.sync_copy(dst_h.at[wid], didx_v)

    def gath(j, _):
        pltpu.async_copy(tbl_s.at[didx_v.at[j]], vals_v.at[j], gsem)
        return 0
    lax.fori_loop(0, CPW, gath, 0)
    pltpu.make_async_copy(src_h.at[wid], vals_v, gsem).wait()

    def scat(j, _):
        pltpu.async_copy(vals_v.at[j], acc_s.at[sidx_v.at[j]], ssem, add=True)
        return 0
    lax.fori_loop(0, CPW, scat, 0)

    def drain(j, _):
        pltpu.make_async_copy(vals_v.at[0], acc_s.at[sidx_v.at[0]], ssem).wait()
        return 0
    lax.fori_loop(0, CPW, drain, 0)
    plsc.subcore_barrier()

    @pl.when(s == 0)
    def _():
        pltpu.sync_copy(acc_s, buf_v)
        pltpu.sync_copy(buf_v, out_h.at[c])


def _norms_body(dop_ref, dip_ref, ns_ref, nd_ref):
    do = dop_ref[0] + dop_ref[1]
    di = dip_ref[0] + dip_ref[1]
    ns_ref[...] = lax.rsqrt(jnp.maximum(do, 1.0))
    nd_ref[...] = lax.rsqrt(jnp.maximum(di, 1.0))


_norms_call = pl.pallas_call(
    _norms_body,
    out_shape=(
        jax.ShapeDtypeStruct((ROWS, 128), F32),
        jax.ShapeDtypeStruct((ROWS, 128), F32),
    ),
)


def _u_body(wp_ref, ns_ref, nd_ref, u_ref, sw_ref):
    w = ns_ref[...] * (wp_ref[0] + wp_ref[1])
    u_ref[...] = nd_ref[...] * w
    r = lax.broadcasted_iota(jnp.int32, (ROWS, 128), 0)
    col = lax.broadcasted_iota(jnp.int32, (ROWS, 128), 1)
    valid = (r * 128 + col) < N_NODES
    sw_ref[...] = jnp.sum(jnp.where(valid, w, 0.0)).reshape(1, 1)


_u_call = pl.pallas_call(
    _u_body,
    out_shape=(
        jax.ShapeDtypeStruct((ROWS, 128), F32),
        jax.ShapeDtypeStruct((1, 1), F32),
    ),
)


BLK = 2000
GRID = N_NODES // BLK


def _final_body(vp0_ref, vp1_ref, ns_ref, x_ref, w1_ref, b1_ref, w2_ref,
                b2_ref, sw_ref, out_ref, acc_ref):
    i = pl.program_id(0)

    @pl.when(i == 0)
    def _():
        acc_ref[...] = jnp.zeros_like(acc_ref)

    v = ns_ref[...] * (vp0_ref[...] + vp1_ref[...])
    acc_ref[...] += jnp.sum(v * x_ref[...], axis=0, keepdims=True)

    @pl.when(i == GRID - 1)
    def _():
        y = acc_ref[...]
        t = jnp.dot(y, w1_ref[...], preferred_element_type=F32)
        t = jnp.dot(t, w2_ref[...], preferred_element_type=F32)
        bias = sw_ref[...] * jnp.dot(b1_ref[...], w2_ref[...],
                                     preferred_element_type=F32)
        out_ref[...] = (t + bias) * (1.0 / N_NODES) + b2_ref[...]


_final_call = pl.pallas_call(
    _final_body,
    grid=(GRID,),
    in_specs=[
        pl.BlockSpec((BLK, 1), lambda i: (i, 0)),
        pl.BlockSpec((BLK, 1), lambda i: (i, 0)),
        pl.BlockSpec((BLK, 1), lambda i: (i, 0)),
        pl.BlockSpec((BLK, 128), lambda i: (i, 0)),
        pl.BlockSpec((128, 128), lambda i: (0, 0)),
        pl.BlockSpec((1, 128), lambda i: (0, 0)),
        pl.BlockSpec((128, 64), lambda i: (0, 0)),
        pl.BlockSpec((1, 64), lambda i: (0, 0)),
        pl.BlockSpec((1, 1), lambda i: (0, 0)),
    ],
    out_specs=pl.BlockSpec((1, 64), lambda i: (0, 0)),
    out_shape=jax.ShapeDtypeStruct((1, 64), F32),
    scratch_shapes=[pltpu.VMEM((1, 128), F32)],
)


def kernel(x, edge_index, W1, b1, W2, b2):
    src = edge_index[0]
    dst = edge_index[1]
    pad = jnp.full((E_PAD - E_EDGES,), N_NODES, jnp.int32)
    src_p = jnp.concatenate([src, pad]).reshape(NUM_WORKERS, CPW, CHUNK)
    dst_p = jnp.concatenate([dst, pad]).reshape(NUM_WORKERS, CPW, CHUNK)

    zeros = jnp.zeros((N_PAD,), F32)
    dop, dip = _deg_kernel(src_p, dst_p, zeros)
    ns, nd = _norms_call(dop.reshape(NUM_CORES, ROWS, 128),
                         dip.reshape(NUM_CORES, ROWS, 128))
    wp = _edge_kernel(src_p, dst_p, nd.reshape(N_PAD), zeros)
    u, sw = _u_call(wp.reshape(NUM_CORES, ROWS, 128), ns, nd)
    vp = _edge_kernel(src_p, dst_p, u.reshape(N_PAD), zeros)

    out = _final_call(
        vp[0].reshape(N_PAD, 1), vp[1].reshape(N_PAD, 1),
        ns.reshape(N_PAD, 1), x, W1, b1.reshape(1, -1), W2,
        b2.reshape(1, -1), sw)
    return out
